# baseline (speedup 1.0000x reference)
import functools

import jax
import jax.numpy as jnp
from jax import lax
from jax.experimental import pallas as pl
from jax.experimental.pallas import tpu as pltpu
from jax.experimental.pallas import tpu_sc as plsc

NW = 32
LANE = 128
K = 4
NBUF = 2


@functools.lru_cache(maxsize=None)
def _build(B, D):
    assert B % (NW * K * LANE) == 0
    nchunk = B // (NW * LANE)
    nbig = nchunk // K
    assert nbig % 2 == 0 and nbig >= 4

    mesh = plsc.VectorSubcoreMesh(core_axis_name="c", subcore_axis_name="s")

    @functools.partial(
        pl.kernel,
        out_type=jax.ShapeDtypeStruct((B, D), jnp.float32),
        mesh=mesh,
        scratch_types=[
            pltpu.VMEM((nchunk * LANE,), jnp.int32),
            pltpu.VMEM((NBUF, K * LANE, D), jnp.float32),
            pltpu.SemaphoreType.DMA((NBUF,)),
            pltpu.SemaphoreType.DMA((NBUF,)),
        ],
        compiler_params=pltpu.CompilerParams(use_tc_tiling_on_sc=False),
    )
    def gather_kernel(table, idx_hbm, out_hbm, idx_v, rows, gsem, ssem):
        w = lax.axis_index("s") * 2 + lax.axis_index("c")
        pltpu.sync_copy(idx_hbm.at[pl.ds(w * nchunk * LANE, nchunk * LANE)],
                        idx_v)
        out_base = w * nbig

        def idx_slice(j):
            return idx_v.at[pl.ds(j * K * LANE, K * LANE)]

        def start_gather(j, b):
            pltpu.async_copy(table.at[idx_slice(j)], rows.at[b], gsem.at[b])

        def wait_gather(b):
            pltpu.make_async_copy(table.at[idx_slice(0)], rows.at[b],
                                  gsem.at[b]).wait()

        def out_slice(j):
            return out_hbm.at[pl.ds((out_base + j) * K * LANE, K * LANE), :]

        def start_scatter(j, b):
            pltpu.async_copy(rows.at[b], out_slice(j), ssem.at[b])

        def wait_scatter(j, b):
            pltpu.make_async_copy(rows.at[b], out_slice(j), ssem.at[b]).wait()

        start_gather(0, 0)
        wait_gather(0)
        start_scatter(0, 0)
        start_gather(1, 1)

        @pl.loop(0, (nbig - 2) // 2)
        def _(g):
            for t in range(2):
                j = 2 * g + 1 + t
                b = (1 + t) % 2
                wait_gather(b)
                start_scatter(j, b)
                wait_scatter(j - 1, 1 - b)
                start_gather(j + 1, 1 - b)

        wait_gather(1)
        start_scatter(nbig - 1, 1)
        wait_scatter(nbig - 2, 0)
        wait_scatter(nbig - 1, 1)

    return gather_kernel


def kernel(image, gather_indices):
    nb, nf, _ = gather_indices.shape
    B = nb * nf
    D = image.shape[1]
    idx = gather_indices.reshape(B).astype(jnp.int32)
    out = _build(B, D)(image, idx)
    return out.reshape(nb, nf, D)


# speedup vs baseline: 1.0627x; 1.0627x over previous
import functools

import jax
import jax.numpy as jnp
from jax import lax
from jax.experimental import pallas as pl
from jax.experimental.pallas import tpu as pltpu
from jax.experimental.pallas import tpu_sc as plsc

NW = 32
LANE = 128
K = 4
NBUF = 2


@functools.lru_cache(maxsize=None)
def _build(B, D):
    assert B % (NW * K * LANE) == 0
    nchunk = B // (NW * LANE)
    nbig = nchunk // K
    assert nbig % 2 == 0 and nbig >= 4

    mesh = plsc.VectorSubcoreMesh(core_axis_name="c", subcore_axis_name="s")

    @functools.partial(
        pl.kernel,
        out_type=jax.ShapeDtypeStruct((B, D), jnp.float32),
        mesh=mesh,
        scratch_types=[
            pltpu.VMEM((nchunk * LANE,), jnp.int32),
            pltpu.VMEM((NBUF, K * LANE, D), jnp.float32),
            pltpu.SemaphoreType.DMA((NBUF,)),
            pltpu.SemaphoreType.DMA((NBUF,)),
        ],
        compiler_params=pltpu.CompilerParams(use_tc_tiling_on_sc=False),
    )
    def gather_kernel(table, idx_hbm, out_hbm, idx_v, rows, gsem, ssem):
        w = lax.axis_index("s") * 2 + lax.axis_index("c")
        pltpu.sync_copy(idx_hbm.at[pl.ds(w * nchunk * LANE, nchunk * LANE)],
                        idx_v)
        out_base = w * nbig

        def idx_slice(j):
            return idx_v.at[pl.ds(j * K * LANE, K * LANE)]

        def start_gather(j, b):
            pltpu.async_copy(table.at[idx_slice(j)], rows.at[b], gsem.at[b])

        def wait_gather(b):
            pltpu.make_async_copy(table.at[idx_slice(0)], rows.at[b],
                                  gsem.at[b]).wait()

        def out_slice(j):
            return out_hbm.at[pl.ds((out_base + j) * K * LANE, K * LANE), :]

        def start_scatter(j, b):
            pltpu.async_copy(rows.at[b], out_slice(j), ssem.at[b])

        def wait_scatter(j, b):
            pltpu.make_async_copy(rows.at[b], out_slice(j), ssem.at[b]).wait()

        start_gather(0, 0)
        wait_gather(0)
        start_scatter(0, 0)
        start_gather(1, 1)

        @pl.loop(0, (nbig - 2) // 2)
        def _(g):
            for t in range(2):
                j = 2 * g + 1 + t
                b = (1 + t) % 2
                wait_gather(b)
                start_scatter(j, b)
                wait_scatter(j - 1, 1 - b)
                start_gather(j + 1, 1 - b)

        wait_gather(1)
        start_scatter(nbig - 1, 1)
        wait_scatter(nbig - 2, 0)
        wait_scatter(nbig - 1, 1)

    return gather_kernel


def kernel(image, gather_indices):
    nb, nf, _ = gather_indices.shape
    B = nb * nf
    D = image.shape[1]
    idx = jnp.transpose(gather_indices, (1, 2, 0)).reshape(B).astype(jnp.int32)
    out = _build(B, D)(image, idx)
    return out.reshape(nf, nb, D).transpose(1, 0, 2)
